# baseline (speedup 1.0000x reference)
import functools
import jax
import jax.numpy as jnp
from jax import lax
from jax.experimental import pallas as pl
from jax.experimental.pallas import tpu as pltpu
from jax.experimental.pallas import tpu_sc as plsc

NV = 10000
EV = 320000
NF = 2000
EF = 16000
EL = 10000
DIN = 128
D = 64

NC = 2
NS = 16
NW = NC * NS
CHUNK = 128
ZC = 128


def _ceil_to(x, m):
    return (x + m - 1) // m * m


def _make_edge_kernel(e_pad, n_table_pad, n_out_pad, has_u, apply_leaky,
                      stage_table=True):
    n_sub = e_pad // (NW * CHUNK)
    ngr = n_sub // 4
    zr = n_out_pad // NS
    ztr = n_table_pad // NS
    assert e_pad % (NW * CHUNK * 4) == 0 and n_out_pad % (NS * ZC) == 0
    assert n_table_pad % (NS * ZC) == 0

    mesh = plsc.VectorSubcoreMesh(core_axis_name="c", subcore_axis_name="s",
                                  num_cores=NC, num_subcores=NS)

    scratch = [
        pltpu.VMEM((n_sub, CHUNK), jnp.int32),
        pltpu.VMEM((n_sub, CHUNK), jnp.int32),
        pltpu.VMEM((CHUNK, D), jnp.float32),
        pltpu.VMEM((CHUNK, D), jnp.float32),
        pltpu.VMEM((CHUNK, D), jnp.float32),
        pltpu.VMEM((CHUNK, D), jnp.float32),
        pltpu.VMEM_SHARED((n_out_pad, D), jnp.float32),
    ] + ([pltpu.VMEM_SHARED((n_table_pad, D), jnp.float32)]
         if stage_table else []) \
      + [pltpu.SemaphoreType.DMA] * 12

    @functools.partial(
        pl.kernel,
        out_type=jax.ShapeDtypeStruct((NC, n_out_pad, D), jnp.float32),
        mesh=mesh,
        scratch_types=scratch,
        compiler_params=pltpu.CompilerParams(use_tc_tiling_on_sc=False),
    )
    def k(*refs):
        if has_u:
            (t_hbm, src_hbm, dst_hbm, u_hbm, zeros_hbm, out_hbm,
             src_all, dst_all, b0, b1, b2, b3, acc, *rest) = refs
        else:
            (t_hbm, src_hbm, dst_hbm, zeros_hbm, out_hbm,
             src_all, dst_all, b0, b1, b2, b3, acc, *rest) = refs
        if stage_table:
            tbl, *sems = rest
        else:
            tbl = t_hbm
            sems = rest
        bufs = (b0, b1, b2, b3)
        sus = sems[0:4]
        sgs = sems[4:8]
        sss = sems[8:12]
        r0 = b0
        r1 = b1

        cid = lax.axis_index("c")
        sid = lax.axis_index("s")
        wid = cid * NS + sid
        base = wid * n_sub

        pltpu.sync_copy(zeros_hbm, r0)
        def zinit(i, _):
            pltpu.sync_copy(r0, acc.at[pl.ds(sid * zr + i * ZC, ZC)])
            return 0
        lax.fori_loop(0, zr // ZC, zinit, 0)
        if stage_table:
            def tinit(i, _):
                trow = sid * ztr + i * ZC
                pltpu.sync_copy(t_hbm.at[pl.ds(trow, ZC)], r1)
                pltpu.sync_copy(r1, tbl.at[pl.ds(trow, ZC)])
                return 0
            lax.fori_loop(0, ztr // ZC, tinit, 0)
        plsc.subcore_barrier()

        pltpu.sync_copy(src_hbm.at[pl.ds(base, n_sub)], src_all)
        pltpu.sync_copy(dst_hbm.at[pl.ds(base, n_sub)], dst_all)

        def issue_u(j, kb):
            pltpu.async_copy(u_hbm.at[pl.ds((base + j) * CHUNK, CHUNK)],
                             bufs[kb], sus[kb])

        def wait_u(j, kb):
            pltpu.make_async_copy(
                u_hbm.at[pl.ds((base + j) * CHUNK, CHUNK)],
                bufs[kb], sus[kb]).wait()

        def issue_g(j, kb):
            pltpu.async_copy(tbl.at[src_all.at[j]], bufs[kb], sgs[kb],
                             add=has_u)

        def wait_g(j, kb):
            pltpu.make_async_copy(tbl.at[src_all.at[j]], bufs[kb],
                                  sgs[kb]).wait()

        def issue_s(j, kb):
            pltpu.async_copy(bufs[kb], acc.at[dst_all.at[j]], sss[kb],
                             add=True)

        def wait_s(j, kb):
            pltpu.make_async_copy(bufs[kb], acc.at[dst_all.at[j]],
                                  sss[kb]).wait()

        def leaky_buf(buf):
            def act(i, _):
                r4 = i * 4
                for rr in range(4):
                    for jj in range(D // 16):
                        v = buf[r4 + rr, pl.ds(jj * 16, 16)]
                        buf[r4 + rr, pl.ds(jj * 16, 16)] = \
                            jnp.maximum(v, 0.1 * v)
                return 0
            lax.fori_loop(0, CHUNK // 4, act, 0)

        if has_u:
            pltpu.sync_copy(u_hbm.at[pl.ds(base * CHUNK, CHUNK)], b0)
        issue_g(0, 0)
        if has_u:
            issue_u(1, 1)
            issue_u(2, 2)

        def group(g, _):
            for kk in range(4):
                j = 4 * g + kk
                kb = kk
                kn = (kk + 1) % 4
                kf = (kk + 2) % 4

                if has_u:
                    if kk < 3:
                        wait_u(j + 1, kn)
                        issue_g(j + 1, kn)
                    else:
                        @pl.when(g < ngr - 1)
                        def _():
                            wait_u(j + 1, kn)
                            issue_g(j + 1, kn)
                else:
                    if kk < 3:
                        @pl.when(g >= 1)
                        def _():
                            wait_s(j - 3, kn)
                        issue_g(j + 1, kn)
                    else:
                        @pl.when(g < ngr - 1)
                        def _():
                            wait_s(j - 3, kn)
                            issue_g(j + 1, kn)

                wait_g(j, kb)
                if apply_leaky:
                    leaky_buf(bufs[kb])
                issue_s(j, kb)

                if has_u:
                    if kk == 0:
                        @pl.when(g >= 1)
                        def _():
                            wait_s(j - 2, kf)
                            issue_u(j + 2, kf)
                    elif kk == 1:
                        @pl.when(g >= 1)
                        def _():
                            wait_s(j - 2, kf)
                        issue_u(j + 2, kf)
                    else:
                        @pl.when(g < ngr - 1)
                        def _():
                            wait_s(j - 2, kf)
                            issue_u(j + 2, kf)
            return 0

        lax.fori_loop(0, ngr, group, 0)
        wait_s(n_sub - 4, 0)
        wait_s(n_sub - 3, 1)
        wait_s(n_sub - 2, 2)
        wait_s(n_sub - 1, 3)
        plsc.subcore_barrier()

        def rb(i, _):
            rbase = sid * zr + i * ZC
            pltpu.sync_copy(acc.at[pl.ds(rbase, ZC)], r0)
            pltpu.sync_copy(r0, out_hbm.at[cid, pl.ds(rbase, ZC)])
            return 0
        lax.fori_loop(0, zr // ZC, rb, 0)

    return k


def _pad_edges(src, dst, e, e_pad, dummy):
    pad = e_pad - e
    src_p = jnp.concatenate([src, jnp.zeros((pad,), jnp.int32)])
    dst_p = jnp.concatenate([dst, jnp.full((pad,), dummy, jnp.int32)])
    return src_p.reshape(e_pad // CHUNK, CHUNK), dst_p.reshape(e_pad // CHUNK, CHUNK)


def _dot(a, b):
    return jnp.dot(a, b, preferred_element_type=jnp.float32)


def _proj_body(x_ref, wp_ref, bp_ref, w2_ref, b2_ref, x0_ref, t0_ref):
    x0 = _dot(x_ref[...], wp_ref[...]) + bp_ref[...]
    x0_ref[...] = x0
    t0_ref[...] = _dot(x0, w2_ref[...]) + b2_ref[...]


def _u_body(ea_ref, w0_ref, w1_ref, u0_ref, u1_ref):
    ea = ea_ref[...]
    u0_ref[...] = _dot(ea, w0_ref[...])
    u1_ref[...] = _dot(ea, w1_ref[...])


def _combine_body(with_next, x_ref, p0_ref, p1_ref, wa_ref, wb_ref, b_ref,
                  *rest):
    agg = p0_ref[...] + p1_ref[...]
    xn = _dot(x_ref[...], wa_ref[...]) + _dot(agg, wb_ref[...]) + b_ref[...]
    if with_next:
        w2_ref, b2_ref, out_ref, t_ref = rest
        out_ref[...] = xn
        t_ref[...] = _dot(xn, w2_ref[...]) + b2_ref[...]
    else:
        (out_ref,) = rest
        out_ref[...] = xn


def _hmsg_body(x_ref, p0_ref, p1_ref, wa_ref, wb_ref, b_ref,
               wx_ref, bx_ref, waa_ref, ba_ref, h_ref):
    agg = p0_ref[...] + p1_ref[...]
    xn = _dot(x_ref[...], wa_ref[...]) + _dot(agg, wb_ref[...]) + b_ref[...]
    gate = jnp.tanh(_dot(xn, waa_ref[...]) + ba_ref[...])
    h_ref[...] = (_dot(xn, wx_ref[...]) + bx_ref[...]) * gate


def _gru_body(p0_ref, p1_ref, wz1_ref, bz1_ref, bz2_ref,
              wr_ref, br_ref, wzg_ref, bzg_ref, wn_ref, bn_ref,
              bhr_ref, bhz_ref, bhn_ref, wpf_ref, bpf_ref,
              w2_ref, b2_ref, y_ref, t_ref):
    mf = p0_ref[...] + p1_ref[...]
    z = jax.nn.sigmoid(_dot(mf, wz1_ref[...]) + bz1_ref[...] + bz2_ref[...])
    h = z * mf
    r = jax.nn.sigmoid(_dot(h, wr_ref[...]) + br_ref[...] + bhr_ref[...])
    zz = jax.nn.sigmoid(_dot(h, wzg_ref[...]) + bzg_ref[...] + bhz_ref[...])
    nn_ = jnp.tanh(_dot(h, wn_ref[...]) + bn_ref[...] + r * bhn_ref[...])
    y = _dot((1.0 - zz) * nn_, wpf_ref[...]) + bpf_ref[...]
    y_ref[...] = y
    t_ref[...] = _dot(y, w2_ref[...]) + b2_ref[...]


def _row_spec(br, d):
    return pl.BlockSpec((br, d), lambda i: (i, 0))


def _full_spec(shape):
    return pl.BlockSpec(shape, lambda i: tuple(0 for _ in shape))


def _call_rows(body, n, br, row_ins, full_ins, n_out):
    grid = (n // br,)
    in_specs = ([_row_spec(br, a.shape[1]) for a in row_ins]
                + [_full_spec(a.shape) for a in full_ins])
    out_shape = [jax.ShapeDtypeStruct((n, D), jnp.float32)] * n_out
    out_specs = [_row_spec(br, D)] * n_out
    res = pl.pallas_call(
        body, grid=grid, in_specs=in_specs,
        out_specs=out_specs, out_shape=out_shape,
    )(*row_ins, *full_ins)
    return res


EV_PAD = _ceil_to(EV, NW * CHUNK * 4)
EL_PAD = _ceil_to(EL, NW * CHUNK * 4)
EF_PAD = _ceil_to(EF, NW * CHUNK * 4)
NV_PAD = _ceil_to(NV + 1, NS * ZC)
NF_PAD = _ceil_to(NF + 1, NS * ZC)

_vert_edge_k = _make_edge_kernel(EV_PAD, NV_PAD, NV_PAD, has_u=True,
                                apply_leaky=True, stage_table=False)
_link_edge_k = _make_edge_kernel(EL_PAD, NV_PAD, NF_PAD, has_u=False,
                                apply_leaky=False, stage_table=False)
_frag_edge_k = _make_edge_kernel(EF_PAD, NF_PAD, NF_PAD, has_u=False,
                                apply_leaky=True, stage_table=False)


@jax.jit
def kernel(vert_x, vert_edge_index, vert_edge_attr, frag_edge_index,
           link_edge_index, link_mask, params):
    p = params
    f32 = jnp.float32
    zeros_c = jnp.zeros((ZC, D), f32)

    def b2d(b):
        return b.reshape(1, D).astype(f32)

    def padrows(a, n):
        return jnp.concatenate([a, jnp.zeros((n - a.shape[0], D), f32)])

    vsrc, vdst = _pad_edges(vert_edge_index[0].astype(jnp.int32),
                            vert_edge_index[1].astype(jnp.int32),
                            EV, EV_PAD, NV)
    lsrc, ldst = _pad_edges(link_edge_index[0].astype(jnp.int32),
                            (link_edge_index[1] - NV).astype(jnp.int32),
                            EL, EL_PAD, NF)
    fsrc, fdst = _pad_edges(frag_edge_index[0].astype(jnp.int32),
                            frag_edge_index[1].astype(jnp.int32),
                            EF, EF_PAD, NF)

    ea_p = jnp.concatenate(
        [vert_edge_attr, jnp.zeros((EV_PAD - EV, vert_edge_attr.shape[1]), f32)])

    u0, u1 = pl.pallas_call(
        _u_body, grid=(EV_PAD // 4096,),
        in_specs=[_row_spec(4096, 16), _full_spec((16, D)), _full_spec((16, D))],
        out_specs=[_row_spec(4096, D)] * 2,
        out_shape=[jax.ShapeDtypeStruct((EV_PAD, D), f32)] * 2,
    )(ea_p, p['v0_u2_W'][D:], p['v1_u2_W'][D:])

    x0, t0 = _call_rows(
        _proj_body, NV, 1000, [vert_x],
        [p['proj_v_W'], b2d(p['proj_v_b']), p['v0_u2_W'][:D], b2d(p['v0_u2_b'])],
        2)

    aggp0 = _vert_edge_k(padrows(t0, NV_PAD), vsrc, vdst, u0, zeros_c)

    x1, t1 = _call_rows(
        functools.partial(_combine_body, True), NV, 1000,
        [x0, aggp0[0, :NV], aggp0[1, :NV]],
        [p['v0_u1_W'][:D], p['v0_u1_W'][D:], b2d(p['v0_u1_b']),
         p['v1_u2_W'][:D], b2d(p['v1_u2_b'])],
        2)

    aggp1 = _vert_edge_k(padrows(t1, NV_PAD), vsrc, vdst, u1, zeros_c)

    (h_msg,) = _call_rows(
        _hmsg_body, NV, 1000,
        [x1, aggp1[0, :NV], aggp1[1, :NV]],
        [p['v1_u1_W'][:D], p['v1_u1_W'][D:], b2d(p['v1_u1_b']),
         p['wx_W'], b2d(p['wx_b']), p['wa_W'][:D], b2d(p['wa_b'])],
        1)

    mfp = _link_edge_k(padrows(h_msg, NV_PAD), lsrc, ldst, zeros_c)

    wih = p['gru_Wih']
    bih = p['gru_bih']
    bhh = p['gru_bhh']
    y, tf0 = _call_rows(
        _gru_body, NF, NF,
        [mfp[0, :NF], mfp[1, :NF]],
        [p['wz1_W'], b2d(p['wz1_b']), b2d(p['wz2_b']),
         wih[:, :D], b2d(bih[:D]), wih[:, D:2 * D], b2d(bih[D:2 * D]),
         wih[:, 2 * D:], b2d(bih[2 * D:]),
         b2d(bhh[:D]), b2d(bhh[D:2 * D]), b2d(bhh[2 * D:]),
         p['proj_f_W'], b2d(p['proj_f_b']),
         p['f0_u2_W'], b2d(p['f0_u2_b'])],
        2)

    aggf0 = _frag_edge_k(padrows(tf0, NF_PAD), fsrc, fdst, zeros_c)

    y1, tf1 = _call_rows(
        functools.partial(_combine_body, True), NF, NF,
        [y, aggf0[0, :NF], aggf0[1, :NF]],
        [p['f0_u1_W'][:D], p['f0_u1_W'][D:], b2d(p['f0_u1_b']),
         p['f1_u2_W'], b2d(p['f1_u2_b'])],
        2)

    aggf1 = _frag_edge_k(padrows(tf1, NF_PAD), fsrc, fdst, zeros_c)

    (out,) = _call_rows(
        functools.partial(_combine_body, False), NF, NF,
        [y1, aggf1[0, :NF], aggf1[1, :NF]],
        [p['f1_u1_W'][:D], p['f1_u1_W'][D:], b2d(p['f1_u1_b'])],
        1)
    return out


# speedup vs baseline: 1.4082x; 1.4082x over previous
import functools
import jax
import jax.numpy as jnp
from jax import lax
from jax.experimental import pallas as pl
from jax.experimental.pallas import tpu as pltpu
from jax.experimental.pallas import tpu_sc as plsc

NV = 10000
EV = 320000
NF = 2000
EF = 16000
EL = 10000
DIN = 128
D = 64

NC = 2
NS = 16
NW = NC * NS
CHUNK = 128
ZC = 128


def _ceil_to(x, m):
    return (x + m - 1) // m * m


def _make_edge_kernel(e_pad, n_table_pad, n_out_pad, has_u, apply_leaky):
    n_sub = e_pad // (NW * CHUNK)
    n_pairs = n_sub // 2
    zr = n_out_pad // NS
    ztr = n_table_pad // NS
    assert e_pad % (NW * CHUNK * 2) == 0 and n_out_pad % (NS * ZC) == 0
    assert n_table_pad % (NS * ZC) == 0

    mesh = plsc.VectorSubcoreMesh(core_axis_name="c", subcore_axis_name="s",
                                  num_cores=NC, num_subcores=NS)

    scratch = [
        pltpu.VMEM((n_sub, CHUNK), jnp.int32),
        pltpu.VMEM((n_sub, CHUNK), jnp.int32),
        pltpu.VMEM((CHUNK, D), jnp.float32),
        pltpu.VMEM((CHUNK, D), jnp.float32),
        pltpu.VMEM_SHARED((n_out_pad, D), jnp.float32),
        pltpu.VMEM_SHARED((n_table_pad, D), jnp.float32),
        pltpu.SemaphoreType.DMA,
        pltpu.SemaphoreType.DMA,
        pltpu.SemaphoreType.DMA,
        pltpu.SemaphoreType.DMA,
    ]

    @functools.partial(
        pl.kernel,
        out_type=jax.ShapeDtypeStruct((NC, n_out_pad, D), jnp.float32),
        mesh=mesh,
        scratch_types=scratch,
        compiler_params=pltpu.CompilerParams(use_tc_tiling_on_sc=False),
    )
    def k(*refs):
        if has_u:
            (t_hbm, src_hbm, dst_hbm, u_hbm, zeros_hbm, out_hbm,
             src_all, dst_all, r0, r1, acc, tbl, su0, su1, sg0, sg1) = refs
        else:
            (t_hbm, src_hbm, dst_hbm, zeros_hbm, out_hbm,
             src_all, dst_all, r0, r1, acc, tbl, su0, su1, sg0, sg1) = refs

        cid = lax.axis_index("c")
        sid = lax.axis_index("s")
        wid = cid * NS + sid
        base = wid * n_sub

        pltpu.sync_copy(zeros_hbm, r0)
        def zinit(i, _):
            pltpu.sync_copy(r0, acc.at[pl.ds(sid * zr + i * ZC, ZC)])
            return 0
        lax.fori_loop(0, zr // ZC, zinit, 0)
        def tinit(i, _):
            trow = sid * ztr + i * ZC
            pltpu.sync_copy(t_hbm.at[pl.ds(trow, ZC)], r1)
            pltpu.sync_copy(r1, tbl.at[pl.ds(trow, ZC)])
            return 0
        lax.fori_loop(0, ztr // ZC, tinit, 0)
        plsc.subcore_barrier()

        pltpu.sync_copy(src_hbm.at[pl.ds(base, n_sub)], src_all)
        pltpu.sync_copy(dst_hbm.at[pl.ds(base, n_sub)], dst_all)

        def issue_u(j, buf, sem):
            pltpu.async_copy(u_hbm.at[pl.ds((base + j) * CHUNK, CHUNK)], buf, sem)

        def wait_u(j, buf, sem):
            pltpu.make_async_copy(
                u_hbm.at[pl.ds((base + j) * CHUNK, CHUNK)], buf, sem).wait()

        def issue_g(j, buf, sem):
            pltpu.async_copy(tbl.at[src_all.at[j]], buf, sem, add=has_u)

        def wait_g(j, buf, sem):
            pltpu.make_async_copy(tbl.at[src_all.at[j]], buf, sem).wait()

        def leaky_buf(buf):
            def act(i, _):
                r4 = i * 4
                for rr in range(4):
                    for jj in range(D // 16):
                        v = buf[r4 + rr, pl.ds(jj * 16, 16)]
                        buf[r4 + rr, pl.ds(jj * 16, 16)] = \
                            jnp.maximum(v, 0.1 * v)
                return 0
            lax.fori_loop(0, CHUNK // 4, act, 0)

        def scatter_buf(j, buf):
            pltpu.sync_copy(buf, acc.at[dst_all.at[j]], add=True)

        if has_u:
            pltpu.sync_copy(u_hbm.at[pl.ds(base * CHUNK, CHUNK)], r0)
        issue_g(0, r0, sg0)
        if has_u:
            issue_u(1, r1, su1)

        def pair(p, _):
            j0 = 2 * p

            if has_u:
                wait_u(j0 + 1, r1, su1)
            issue_g(j0 + 1, r1, sg1)
            wait_g(j0, r0, sg0)
            if apply_leaky:
                leaky_buf(r0)
            scatter_buf(j0, r0)
            if has_u:
                @pl.when(p < n_pairs - 1)
                def _():
                    issue_u(j0 + 2, r0, su0)

            @pl.when(p < n_pairs - 1)
            def _():
                if has_u:
                    wait_u(j0 + 2, r0, su0)
                issue_g(j0 + 2, r0, sg0)
            wait_g(j0 + 1, r1, sg1)
            if apply_leaky:
                leaky_buf(r1)
            scatter_buf(j0 + 1, r1)
            if has_u:
                @pl.when(p < n_pairs - 1)
                def _():
                    issue_u(j0 + 3, r1, su1)
            return 0

        lax.fori_loop(0, n_pairs, pair, 0)
        plsc.subcore_barrier()

        def rb(i, _):
            rbase = sid * zr + i * ZC
            pltpu.sync_copy(acc.at[pl.ds(rbase, ZC)], r0)
            pltpu.sync_copy(r0, out_hbm.at[cid, pl.ds(rbase, ZC)])
            return 0
        lax.fori_loop(0, zr // ZC, rb, 0)

    return k


def _pad_edges(src, dst, e, e_pad, dummy):
    pad = e_pad - e
    src_p = jnp.concatenate([src, jnp.zeros((pad,), jnp.int32)])
    dst_p = jnp.concatenate([dst, jnp.full((pad,), dummy, jnp.int32)])
    return src_p.reshape(e_pad // CHUNK, CHUNK), dst_p.reshape(e_pad // CHUNK, CHUNK)


def _dot(a, b):
    return jnp.dot(a, b, preferred_element_type=jnp.float32)


def _proj_body(x_ref, wp_ref, bp_ref, w2_ref, b2_ref, x0_ref, t0_ref):
    x0 = _dot(x_ref[...], wp_ref[...]) + bp_ref[...]
    x0_ref[...] = x0
    t0_ref[...] = _dot(x0, w2_ref[...]) + b2_ref[...]


def _u_body(ea_ref, w0_ref, w1_ref, u0_ref, u1_ref):
    ea = ea_ref[...]
    u0_ref[...] = _dot(ea, w0_ref[...])
    u1_ref[...] = _dot(ea, w1_ref[...])


def _combine_body(with_next, x_ref, p0_ref, p1_ref, wa_ref, wb_ref, b_ref,
                  *rest):
    agg = p0_ref[...] + p1_ref[...]
    xn = _dot(x_ref[...], wa_ref[...]) + _dot(agg, wb_ref[...]) + b_ref[...]
    if with_next:
        w2_ref, b2_ref, out_ref, t_ref = rest
        out_ref[...] = xn
        t_ref[...] = _dot(xn, w2_ref[...]) + b2_ref[...]
    else:
        (out_ref,) = rest
        out_ref[...] = xn


def _hmsg_body(x_ref, p0_ref, p1_ref, wa_ref, wb_ref, b_ref,
               wx_ref, bx_ref, waa_ref, ba_ref, h_ref):
    agg = p0_ref[...] + p1_ref[...]
    xn = _dot(x_ref[...], wa_ref[...]) + _dot(agg, wb_ref[...]) + b_ref[...]
    gate = jnp.tanh(_dot(xn, waa_ref[...]) + ba_ref[...])
    h_ref[...] = (_dot(xn, wx_ref[...]) + bx_ref[...]) * gate


def _gru_body(p0_ref, p1_ref, wz1_ref, bz1_ref, bz2_ref,
              wr_ref, br_ref, wzg_ref, bzg_ref, wn_ref, bn_ref,
              bhr_ref, bhz_ref, bhn_ref, wpf_ref, bpf_ref,
              w2_ref, b2_ref, y_ref, t_ref):
    mf = p0_ref[...] + p1_ref[...]
    z = jax.nn.sigmoid(_dot(mf, wz1_ref[...]) + bz1_ref[...] + bz2_ref[...])
    h = z * mf
    r = jax.nn.sigmoid(_dot(h, wr_ref[...]) + br_ref[...] + bhr_ref[...])
    zz = jax.nn.sigmoid(_dot(h, wzg_ref[...]) + bzg_ref[...] + bhz_ref[...])
    nn_ = jnp.tanh(_dot(h, wn_ref[...]) + bn_ref[...] + r * bhn_ref[...])
    y = _dot((1.0 - zz) * nn_, wpf_ref[...]) + bpf_ref[...]
    y_ref[...] = y
    t_ref[...] = _dot(y, w2_ref[...]) + b2_ref[...]


def _row_spec(br, d):
    return pl.BlockSpec((br, d), lambda i: (i, 0))


def _full_spec(shape):
    return pl.BlockSpec(shape, lambda i: tuple(0 for _ in shape))


def _call_rows(body, n, br, row_ins, full_ins, n_out):
    grid = (n // br,)
    in_specs = ([_row_spec(br, a.shape[1]) for a in row_ins]
                + [_full_spec(a.shape) for a in full_ins])
    out_shape = [jax.ShapeDtypeStruct((n, D), jnp.float32)] * n_out
    out_specs = [_row_spec(br, D)] * n_out
    res = pl.pallas_call(
        body, grid=grid, in_specs=in_specs,
        out_specs=out_specs, out_shape=out_shape,
    )(*row_ins, *full_ins)
    return res


EV_PAD = _ceil_to(EV, NW * CHUNK * 2)
EL_PAD = _ceil_to(EL, NW * CHUNK * 2)
EF_PAD = _ceil_to(EF, NW * CHUNK * 2)
NV_PAD = _ceil_to(NV + 1, NS * ZC)
NF_PAD = _ceil_to(NF + 1, NS * ZC)

_vert_edge_k = _make_edge_kernel(EV_PAD, NV_PAD, NV_PAD, has_u=True, apply_leaky=True)
_link_edge_k = _make_edge_kernel(EL_PAD, NV_PAD, NF_PAD, has_u=False, apply_leaky=False)
_frag_edge_k = _make_edge_kernel(EF_PAD, NF_PAD, NF_PAD, has_u=False, apply_leaky=True)


@jax.jit
def kernel(vert_x, vert_edge_index, vert_edge_attr, frag_edge_index,
           link_edge_index, link_mask, params):
    p = params
    f32 = jnp.float32
    zeros_c = jnp.zeros((ZC, D), f32)

    def b2d(b):
        return b.reshape(1, D).astype(f32)

    def padrows(a, n):
        return jnp.concatenate([a, jnp.zeros((n - a.shape[0], D), f32)])

    vsrc, vdst = _pad_edges(vert_edge_index[0].astype(jnp.int32),
                            vert_edge_index[1].astype(jnp.int32),
                            EV, EV_PAD, NV)
    lsrc, ldst = _pad_edges(link_edge_index[0].astype(jnp.int32),
                            (link_edge_index[1] - NV).astype(jnp.int32),
                            EL, EL_PAD, NF)
    fsrc, fdst = _pad_edges(frag_edge_index[0].astype(jnp.int32),
                            frag_edge_index[1].astype(jnp.int32),
                            EF, EF_PAD, NF)

    ea_p = jnp.concatenate(
        [vert_edge_attr, jnp.zeros((EV_PAD - EV, vert_edge_attr.shape[1]), f32)])

    u0, u1 = pl.pallas_call(
        _u_body, grid=(EV_PAD // 4096,),
        in_specs=[_row_spec(4096, 16), _full_spec((16, D)), _full_spec((16, D))],
        out_specs=[_row_spec(4096, D)] * 2,
        out_shape=[jax.ShapeDtypeStruct((EV_PAD, D), f32)] * 2,
    )(ea_p, p['v0_u2_W'][D:], p['v1_u2_W'][D:])

    x0, t0 = _call_rows(
        _proj_body, NV, 1000, [vert_x],
        [p['proj_v_W'], b2d(p['proj_v_b']), p['v0_u2_W'][:D], b2d(p['v0_u2_b'])],
        2)

    aggp0 = _vert_edge_k(padrows(t0, NV_PAD), vsrc, vdst, u0, zeros_c)

    x1, t1 = _call_rows(
        functools.partial(_combine_body, True), NV, 1000,
        [x0, aggp0[0, :NV], aggp0[1, :NV]],
        [p['v0_u1_W'][:D], p['v0_u1_W'][D:], b2d(p['v0_u1_b']),
         p['v1_u2_W'][:D], b2d(p['v1_u2_b'])],
        2)

    aggp1 = _vert_edge_k(padrows(t1, NV_PAD), vsrc, vdst, u1, zeros_c)

    (h_msg,) = _call_rows(
        _hmsg_body, NV, 1000,
        [x1, aggp1[0, :NV], aggp1[1, :NV]],
        [p['v1_u1_W'][:D], p['v1_u1_W'][D:], b2d(p['v1_u1_b']),
         p['wx_W'], b2d(p['wx_b']), p['wa_W'][:D], b2d(p['wa_b'])],
        1)

    mfp = _link_edge_k(padrows(h_msg, NV_PAD), lsrc, ldst, zeros_c)

    wih = p['gru_Wih']
    bih = p['gru_bih']
    bhh = p['gru_bhh']
    y, tf0 = _call_rows(
        _gru_body, NF, NF,
        [mfp[0, :NF], mfp[1, :NF]],
        [p['wz1_W'], b2d(p['wz1_b']), b2d(p['wz2_b']),
         wih[:, :D], b2d(bih[:D]), wih[:, D:2 * D], b2d(bih[D:2 * D]),
         wih[:, 2 * D:], b2d(bih[2 * D:]),
         b2d(bhh[:D]), b2d(bhh[D:2 * D]), b2d(bhh[2 * D:]),
         p['proj_f_W'], b2d(p['proj_f_b']),
         p['f0_u2_W'], b2d(p['f0_u2_b'])],
        2)

    aggf0 = _frag_edge_k(padrows(tf0, NF_PAD), fsrc, fdst, zeros_c)

    y1, tf1 = _call_rows(
        functools.partial(_combine_body, True), NF, NF,
        [y, aggf0[0, :NF], aggf0[1, :NF]],
        [p['f0_u1_W'][:D], p['f0_u1_W'][D:], b2d(p['f0_u1_b']),
         p['f1_u2_W'], b2d(p['f1_u2_b'])],
        2)

    aggf1 = _frag_edge_k(padrows(tf1, NF_PAD), fsrc, fdst, zeros_c)

    (out,) = _call_rows(
        functools.partial(_combine_body, False), NF, NF,
        [y1, aggf1[0, :NF], aggf1[1, :NF]],
        [p['f1_u1_W'][:D], p['f1_u1_W'][D:], b2d(p['f1_u1_b'])],
        1)
    return out
